# baseline (speedup 1.0000x reference)
import functools

import jax
import jax.numpy as jnp
from jax import lax
from jax.experimental import pallas as pl
from jax.experimental.pallas import tpu as pltpu
from jax.experimental.pallas import tpu_sc as plsc

_NC = 2
_NS = 16
_NW = _NC * _NS
_BBLK = 128
_PITCH = 129


_VBLK = 128
_NFULL = 244
_VPAD = 1003520


def _sc_detile(wt, tailp):
    d_model, vocab = wt.shape
    mesh = plsc.VectorSubcoreMesh(core_axis_name="c", subcore_axis_name="s")

    @functools.partial(
        pl.kernel,
        mesh=mesh,
        out_type=jax.ShapeDtypeStruct((_VPAD, 128), jnp.float32),
        scratch_types=[
            pltpu.VMEM((2, d_model, _VBLK), jnp.float32),
            pltpu.VMEM((2, _VBLK, _PITCH), jnp.float32),
            pltpu.SemaphoreType.DMA((2,)),
            pltpu.SemaphoreType.DMA((2,)),
        ],
        compiler_params=pltpu.CompilerParams(use_tc_tiling_on_sc=True,
                                             needs_layout_passes=False),
    )
    def k(wt_hbm, tail_hbm, out_hbm, slab_v, slab_t, gsem, ssem):
        wid = lax.axis_index("s") * _NC + lax.axis_index("c")

        def fire_load(i, s):
            v0 = (wid + _NW * i) * _VBLK
            pltpu.async_copy(wt_hbm.at[:, pl.ds(v0, _VBLK)], slab_v.at[s],
                             gsem.at[s])

        def drain_load(s):
            pltpu.make_async_copy(wt_hbm.at[:, pl.ds(0, _VBLK)],
                                  slab_v.at[s], gsem.at[s]).wait()

        def fire_store(i, s):
            v0 = (wid + _NW * i) * _VBLK
            pltpu.async_copy(slab_t.at[s, :, pl.ds(0, 128)],
                             out_hbm.at[pl.ds(v0, _VBLK)], ssem.at[s])

        def drain_store(s):
            pltpu.make_async_copy(slab_t.at[s, :, pl.ds(0, 128)],
                                  out_hbm.at[pl.ds(0, _VBLK)],
                                  ssem.at[s]).wait()

        iota = lax.iota(jnp.int32, 16)

        def transpose(s, vchunks):
            @plsc.parallel_loop(0, d_model, unroll=8)
            def _(d):
                cd = jnp.full((16,), 0, jnp.int32) + d
                for vc in range(vchunks):
                    v = slab_v[s, d, pl.ds(vc * 16, 16)]
                    plsc.store_scatter(slab_t.at[s], [vc * 16 + iota, cd], v)

        fire_load(0, 0)

        def outer(i2, carry):
            for p in range(2):
                i = i2 * 2 + p
                cur, nxt = p, 1 - p

                @pl.when(i + 1 < _NFULL)
                def _():
                    fire_load(i + 1, nxt)

                drain_load(cur)

                @pl.when(i >= 2)
                def _():
                    drain_store(cur)

                transpose(cur, _VBLK // 16)
                fire_store(i, cur)
            return carry

        lax.fori_loop(0, _NFULL // 2, outer, 0)
        drain_store(0)
        drain_store(1)

        n_rr = _NW * _NFULL

        @pl.when(wid < 4)
        def _():
            v0 = (n_rr + wid) * _VBLK
            pltpu.sync_copy(wt_hbm.at[:, pl.ds(v0, _VBLK)], slab_v.at[0])
            transpose(0, _VBLK // 16)
            pltpu.sync_copy(slab_t.at[0, :, pl.ds(0, 128)],
                            out_hbm.at[pl.ds(v0, _VBLK)])

        @pl.when(wid == 4)
        def _():
            v0 = (n_rr + 4) * _VBLK
            pltpu.sync_copy(tail_hbm, slab_v.at[0])
            transpose(0, _VBLK // 16)
            pltpu.sync_copy(slab_t.at[0, :, pl.ds(0, 128)],
                            out_hbm.at[pl.ds(v0, _VBLK)])

    return k(wt, tailp)


def _sc_gather_t(table, xt):
    hist, batch = xt.shape
    d_model = table.shape[1]
    mesh = plsc.VectorSubcoreMesh(core_axis_name="c", subcore_axis_name="s")

    @functools.partial(
        pl.kernel,
        mesh=mesh,
        out_type=jax.ShapeDtypeStruct((hist, d_model, batch), jnp.float32),
        scratch_types=[
            pltpu.VMEM((hist, _BBLK), jnp.int32),
            pltpu.VMEM((2, _BBLK, d_model), jnp.float32),
            pltpu.VMEM((2, d_model, _PITCH), jnp.float32),
            pltpu.SemaphoreType.DMA((2,)),
            pltpu.SemaphoreType.DMA((2,)),
        ],
        compiler_params=pltpu.CompilerParams(use_tc_tiling_on_sc=False,
                                             needs_layout_passes=False),
    )
    def k(table_hbm, xt_hbm, out_hbm, idx_v, rows_v, rows_t, gsem, ssem):
        wid = lax.axis_index("s") * _NC + lax.axis_index("c")
        col0 = wid * _BBLK
        pltpu.sync_copy(xt_hbm.at[:, pl.ds(col0, _BBLK)], idx_v)

        def fire_gather(h, s):
            pltpu.async_copy(table_hbm.at[idx_v.at[h]], rows_v.at[s],
                             gsem.at[s])

        def drain_gather(s):
            pltpu.make_async_copy(table_hbm.at[idx_v.at[0]], rows_v.at[s],
                                  gsem.at[s]).wait()

        def fire_store(h, s):
            pltpu.async_copy(rows_t.at[s, :, pl.ds(0, _BBLK)],
                             out_hbm.at[h, :, pl.ds(col0, _BBLK)],
                             ssem.at[s])

        def drain_store(s):
            pltpu.make_async_copy(rows_t.at[s, :, pl.ds(0, _BBLK)],
                                  out_hbm.at[0, :, pl.ds(col0, _BBLK)],
                                  ssem.at[s]).wait()

        iota = lax.iota(jnp.int32, 16)

        def transpose(s):
            @plsc.parallel_loop(0, _BBLK, unroll=8)
            def _(b):
                cb = jnp.full((16,), 0, jnp.int32) + b
                for dc in range(d_model // 16):
                    v = rows_v[s, b, pl.ds(dc * 16, 16)]
                    plsc.store_scatter(rows_t.at[s],
                                       [dc * 16 + iota, cb], v)

        fire_gather(0, 0)

        def outer(i, carry):
            for p in range(2):
                h = i * 2 + p
                cur, nxt = p, 1 - p

                @pl.when(h + 1 < hist)
                def _():
                    fire_gather(h + 1, nxt)

                drain_gather(cur)

                @pl.when(h >= 2)
                def _():
                    drain_store(cur)

                transpose(cur)
                fire_store(h, cur)
            return carry

        lax.fori_loop(0, hist // 2, outer, 0)
        drain_store(0)
        drain_store(1)

    return k(table, xt)


def kernel(x, weights):
    wt = jnp.transpose(weights)
    tail = wt[:, (_NW * _NFULL + 4) * _VBLK:]
    tailp = jnp.pad(tail, ((0, 0), (0, _VBLK - tail.shape[1])))
    table_pad = _sc_detile(wt, tailp)
    table2 = table_pad.reshape(2 * _VPAD, 64)
    xt = jnp.transpose(x).astype(jnp.int32) * 2
    out_t = _sc_gather_t(table2, xt)
    return jnp.transpose(out_t, (2, 0, 1))


# speedup vs baseline: 1.0070x; 1.0070x over previous
import functools

import jax
import jax.numpy as jnp
from jax import lax
from jax.experimental import pallas as pl
from jax.experimental.pallas import tpu as pltpu
from jax.experimental.pallas import tpu_sc as plsc

_NC = 2
_NS = 16
_NW = _NC * _NS
_BBLK = 128
_PITCH = 129


_VBLK = 128
_NFULL = 244
_VPAD = 1003520


def _sc_detile(wt, tailp):
    d_model, vocab = wt.shape
    mesh = plsc.VectorSubcoreMesh(core_axis_name="c", subcore_axis_name="s")

    @functools.partial(
        pl.kernel,
        mesh=mesh,
        out_type=jax.ShapeDtypeStruct((_VPAD // 2, 128), jnp.float32),
        scratch_types=[
            pltpu.VMEM((4, d_model, _VBLK), jnp.float32),
            pltpu.VMEM((2, d_model, _PITCH), jnp.float32),
            pltpu.SemaphoreType.DMA((2,)),
            pltpu.SemaphoreType.DMA((2,)),
        ],
        compiler_params=pltpu.CompilerParams(use_tc_tiling_on_sc=True,
                                             needs_layout_passes=False),
    )
    def k(wt_hbm, tail_hbm, out_hbm, slab_v, slab_t, gsem, ssem):
        wid = lax.axis_index("s") * _NC + lax.axis_index("c")

        def fire_load(i, s):
            v0 = (wid + _NW * i) * _VBLK
            pltpu.async_copy(wt_hbm.at[:, pl.ds(v0, _VBLK)], slab_v.at[s],
                             gsem.at[s])

        def drain_load(s):
            pltpu.make_async_copy(wt_hbm.at[:, pl.ds(0, _VBLK)],
                                  slab_v.at[s], gsem.at[s]).wait()

        def fire_store(i, s):
            r0 = (wid + _NW * i) * (_VBLK // 2)
            pltpu.async_copy(slab_t.at[s, :, pl.ds(0, 128)],
                             out_hbm.at[pl.ds(r0, _VBLK // 2)], ssem.at[s])

        def drain_store(s):
            pltpu.make_async_copy(slab_t.at[s, :, pl.ds(0, 128)],
                                  out_hbm.at[pl.ds(0, _VBLK // 2)],
                                  ssem.at[s]).wait()

        iota = lax.iota(jnp.int32, 16)

        def transpose(sv, st, vchunks):
            @plsc.parallel_loop(0, d_model, unroll=8)
            def _(d):
                cd = jnp.full((16,), 0, jnp.int32) + d
                rbase = iota >> 1
                cbase = (iota & 1) * d_model
                for vc in range(vchunks):
                    v = slab_v[sv, d, pl.ds(vc * 16, 16)]
                    plsc.store_scatter(slab_t.at[st],
                                       [vc * 8 + rbase, cbase + cd], v)

        fire_load(0, 0)
        fire_load(1, 1)
        fire_load(2, 2)

        def outer(i4, carry):
            for p in range(4):
                i = i4 * 4 + p
                st = p % 2

                @pl.when(i + 3 < _NFULL)
                def _():
                    fire_load(i + 3, (p + 3) % 4)

                drain_load(p)

                @pl.when(i >= 2)
                def _():
                    drain_store(st)

                transpose(p, st, _VBLK // 16)
                fire_store(i, st)
            return carry

        lax.fori_loop(0, _NFULL // 4, outer, 0)
        drain_store(0)
        drain_store(1)

        n_rr = _NW * _NFULL

        @pl.when(wid < 4)
        def _():
            v0 = (n_rr + wid) * _VBLK
            pltpu.sync_copy(wt_hbm.at[:, pl.ds(v0, _VBLK)], slab_v.at[0])
            transpose(0, 0, _VBLK // 16)
            pltpu.sync_copy(slab_t.at[0, :, pl.ds(0, 128)],
                            out_hbm.at[pl.ds((n_rr + wid) * (_VBLK // 2),
                                             _VBLK // 2)])

        @pl.when(wid == 4)
        def _():
            pltpu.sync_copy(tail_hbm, slab_v.at[0])
            transpose(0, 0, _VBLK // 16)
            pltpu.sync_copy(slab_t.at[0, :, pl.ds(0, 128)],
                            out_hbm.at[pl.ds((n_rr + 4) * (_VBLK // 2),
                                             _VBLK // 2)])

    return k(wt, tailp)


def _sc_gather_t(table, xt):
    hist, batch = xt.shape
    d_model = table.shape[1]
    mesh = plsc.VectorSubcoreMesh(core_axis_name="c", subcore_axis_name="s")

    @functools.partial(
        pl.kernel,
        mesh=mesh,
        out_type=jax.ShapeDtypeStruct((hist, d_model, batch), jnp.float32),
        scratch_types=[
            pltpu.VMEM((hist, _BBLK), jnp.int32),
            pltpu.VMEM((2, _BBLK, d_model), jnp.float32),
            pltpu.VMEM((2, d_model, _PITCH), jnp.float32),
            pltpu.SemaphoreType.DMA((2,)),
            pltpu.SemaphoreType.DMA((2,)),
        ],
        compiler_params=pltpu.CompilerParams(use_tc_tiling_on_sc=False,
                                             needs_layout_passes=False),
    )
    def k(table_hbm, xt_hbm, out_hbm, idx_v, rows_v, rows_t, gsem, ssem):
        wid = lax.axis_index("s") * _NC + lax.axis_index("c")
        col0 = wid * _BBLK
        pltpu.sync_copy(xt_hbm.at[:, pl.ds(col0, _BBLK)], idx_v)

        def fire_gather(h, s):
            pltpu.async_copy(table_hbm.at[idx_v.at[h]], rows_v.at[s],
                             gsem.at[s])

        def drain_gather(s):
            pltpu.make_async_copy(table_hbm.at[idx_v.at[0]], rows_v.at[s],
                                  gsem.at[s]).wait()

        def fire_store(h, s):
            pltpu.async_copy(rows_t.at[s, :, pl.ds(0, _BBLK)],
                             out_hbm.at[h, :, pl.ds(col0, _BBLK)],
                             ssem.at[s])

        def drain_store(s):
            pltpu.make_async_copy(rows_t.at[s, :, pl.ds(0, _BBLK)],
                                  out_hbm.at[0, :, pl.ds(col0, _BBLK)],
                                  ssem.at[s]).wait()

        iota = lax.iota(jnp.int32, 16)

        def transpose(s):
            @plsc.parallel_loop(0, _BBLK, unroll=8)
            def _(b):
                cb = jnp.full((16,), 0, jnp.int32) + b
                for dc in range(d_model // 16):
                    v = rows_v[s, b, pl.ds(dc * 16, 16)]
                    plsc.store_scatter(rows_t.at[s],
                                       [dc * 16 + iota, cb], v)

        fire_gather(0, 0)

        def outer(i, carry):
            for p in range(2):
                h = i * 2 + p
                cur, nxt = p, 1 - p

                @pl.when(h + 1 < hist)
                def _():
                    fire_gather(h + 1, nxt)

                drain_gather(cur)

                @pl.when(h >= 2)
                def _():
                    drain_store(cur)

                transpose(cur)
                fire_store(h, cur)
            return carry

        lax.fori_loop(0, hist // 2, outer, 0)
        drain_store(0)
        drain_store(1)

    return k(table, xt)


def kernel(x, weights):
    wt = jnp.transpose(weights)
    tail = wt[:, (_NW * _NFULL + 4) * _VBLK:]
    tailp = jnp.pad(tail, ((0, 0), (0, _VBLK - tail.shape[1])))
    table_pad = _sc_detile(wt, tailp)
    table2 = table_pad.reshape(_VPAD, 64)
    xt = jnp.transpose(x).astype(jnp.int32)
    out_t = _sc_gather_t(table2, xt)
    return jnp.transpose(out_t, (2, 0, 1))


# speedup vs baseline: 1.2748x; 1.2659x over previous
import functools

import jax
import jax.numpy as jnp
from jax import lax
from jax.experimental import pallas as pl
from jax.experimental.pallas import tpu as pltpu
from jax.experimental.pallas import tpu_sc as plsc

_NC = 2
_NS = 16
_NW = _NC * _NS
_BBLK = 128
_PITCH = 129


def _sc_gather_t(table, xt):
    hist, batch = xt.shape
    d_model = table.shape[1]
    mesh = plsc.VectorSubcoreMesh(core_axis_name="c", subcore_axis_name="s")

    @functools.partial(
        pl.kernel,
        mesh=mesh,
        out_type=jax.ShapeDtypeStruct((hist, d_model, batch), jnp.float32),
        scratch_types=[
            pltpu.VMEM((hist, _BBLK), jnp.int32),
            pltpu.VMEM((2, _BBLK, d_model), jnp.float32),
            pltpu.VMEM((2, d_model, _PITCH), jnp.float32),
            pltpu.SemaphoreType.DMA((2,)),
            pltpu.SemaphoreType.DMA((2,)),
        ],
        compiler_params=pltpu.CompilerParams(use_tc_tiling_on_sc=False,
                                             needs_layout_passes=False),
    )
    def k(table_hbm, xt_hbm, out_hbm, idx_v, rows_v, rows_t, gsem, ssem):
        wid = lax.axis_index("s") * _NC + lax.axis_index("c")
        col0 = wid * _BBLK
        pltpu.sync_copy(xt_hbm.at[:, pl.ds(col0, _BBLK)], idx_v)

        def fire_gather(h, s):
            pltpu.async_copy(table_hbm.at[idx_v.at[h]], rows_v.at[s],
                             gsem.at[s])

        def drain_gather(s):
            pltpu.make_async_copy(table_hbm.at[idx_v.at[0]], rows_v.at[s],
                                  gsem.at[s]).wait()

        def fire_store(h, s):
            pltpu.async_copy(rows_t.at[s, :, pl.ds(0, _BBLK)],
                             out_hbm.at[h, :, pl.ds(col0, _BBLK)],
                             ssem.at[s])

        def drain_store(s):
            pltpu.make_async_copy(rows_t.at[s, :, pl.ds(0, _BBLK)],
                                  out_hbm.at[0, :, pl.ds(col0, _BBLK)],
                                  ssem.at[s]).wait()

        iota = lax.iota(jnp.int32, 16)

        def transpose(s):
            @plsc.parallel_loop(0, _BBLK, unroll=8)
            def _(b):
                cb = jnp.full((16,), 0, jnp.int32) + b
                for dc in range(d_model // 16):
                    v = rows_v[s, b, pl.ds(dc * 16, 16)]
                    plsc.store_scatter(rows_t.at[s],
                                       [dc * 16 + iota, cb], v)

        fire_gather(0, 0)

        def outer(i, carry):
            for p in range(2):
                h = i * 2 + p
                cur, nxt = p, 1 - p

                @pl.when(h + 1 < hist)
                def _():
                    fire_gather(h + 1, nxt)

                drain_gather(cur)

                @pl.when(h >= 2)
                def _():
                    drain_store(cur)

                transpose(cur)
                fire_store(h, cur)
            return carry

        lax.fori_loop(0, hist // 2, outer, 0)
        drain_store(0)
        drain_store(1)

    return k(table, xt)


def kernel(x, weights):
    xt = jnp.transpose(x).astype(jnp.int32)
    out_t = _sc_gather_t(weights, xt)
    return jnp.transpose(out_t, (2, 0, 1))
